# baseline (speedup 1.0000x reference)
import functools

import jax
import jax.numpy as jnp
from jax.experimental import pallas as pl
from jax.experimental.pallas import tpu as pltpu

_LN_EPS = 1e-5


def _moe_kernel(x_ref, w_ref, gw_ref, gamma_ref, beta_ref, o_ref,
                lg_ref, *, k_blocks, num_experts):
    k = pl.program_id(1)

    @pl.when(k == 0)
    def _():
        o_ref[...] = jnp.zeros_like(o_ref)
        lg_ref[...] = jnp.zeros_like(lg_ref)

    x = x_ref[...]
    o_ref[...] += jnp.dot(x, w_ref[...], preferred_element_type=jnp.float32)
    lg_ref[...] += jax.lax.dot_general(
        x, gw_ref[...], (((1,), (1,)), ((), ())),
        preferred_element_type=jnp.float32)

    @pl.when(k == k_blocks - 1)
    def _():
        logits = lg_ref[...]
        m1 = jnp.max(logits, axis=-1, keepdims=True)
        iota = jax.lax.broadcasted_iota(jnp.int32, logits.shape, 1)
        is_max = logits == m1
        first_idx = jnp.min(jnp.where(is_max, iota, num_experts),
                            axis=-1, keepdims=True)
        masked = jnp.where(iota == first_idx, -jnp.inf, logits)
        m2 = jnp.max(masked, axis=-1, keepdims=True)
        s = m1 + m2

        moe = o_ref[...] * s
        mean = jnp.mean(moe, axis=-1, keepdims=True)
        var = jnp.mean(jnp.square(moe - mean), axis=-1, keepdims=True)
        o_ref[...] = ((moe - mean) * jax.lax.rsqrt(var + _LN_EPS)
                      * gamma_ref[...] + beta_ref[...])


@jax.jit
def kernel(hidden_states, gate_w, expert_weight, ln_gamma, ln_beta):
    b, s, h = hidden_states.shape
    e = gate_w.shape[0]
    m = b * s
    bm = min(512, m)
    bk = min(512, h)
    m_blocks = m // bm
    k_blocks = h // bk

    x2d = hidden_states.reshape(m, h)
    gamma2d = ln_gamma.reshape(1, h)
    beta2d = ln_beta.reshape(1, h)

    out = pl.pallas_call(
        functools.partial(_moe_kernel, k_blocks=k_blocks, num_experts=e),
        grid=(m_blocks, k_blocks),
        in_specs=[
            pl.BlockSpec((bm, bk), lambda i, k: (i, k)),
            pl.BlockSpec((bk, h), lambda i, k: (k, 0)),
            pl.BlockSpec((e, bk), lambda i, k: (0, k)),
            pl.BlockSpec((1, h), lambda i, k: (0, 0)),
            pl.BlockSpec((1, h), lambda i, k: (0, 0)),
        ],
        out_specs=pl.BlockSpec((bm, h), lambda i, k: (i, 0)),
        out_shape=jax.ShapeDtypeStruct((m, h), jnp.float32),
        scratch_shapes=[
            pltpu.VMEM((bm, e), jnp.float32),
        ],
        compiler_params=pltpu.CompilerParams(
            dimension_semantics=("parallel", "arbitrary")),
    )(x2d, expert_weight, gate_w, gamma2d, beta2d)

    return out.reshape(b, s, h)


# speedup vs baseline: 1.0629x; 1.0629x over previous
import functools

import jax
import jax.numpy as jnp
from jax.experimental import pallas as pl
from jax.experimental.pallas import tpu as pltpu

_LN_EPS = 1e-5


def _moe_kernel(x_ref, w_ref, gw_ref, gamma_ref, beta_ref, o_ref,
                *, num_experts):
    x = x_ref[...]
    acc = jnp.dot(x, w_ref[...], preferred_element_type=jnp.float32)
    logits = jax.lax.dot_general(
        x, gw_ref[...], (((1,), (1,)), ((), ())),
        preferred_element_type=jnp.float32)

    m1 = jnp.max(logits, axis=-1, keepdims=True)
    iota = jax.lax.broadcasted_iota(jnp.int32, logits.shape, 1)
    is_max = logits == m1
    first_idx = jnp.min(jnp.where(is_max, iota, num_experts),
                        axis=-1, keepdims=True)
    masked = jnp.where(iota == first_idx, -jnp.inf, logits)
    m2 = jnp.max(masked, axis=-1, keepdims=True)
    s = m1 + m2

    moe = acc * s
    mean = jnp.mean(moe, axis=-1, keepdims=True)
    var = jnp.mean(jnp.square(moe - mean), axis=-1, keepdims=True)
    o_ref[...] = ((moe - mean) * jax.lax.rsqrt(var + _LN_EPS)
                  * gamma_ref[...] + beta_ref[...])


@jax.jit
def kernel(hidden_states, gate_w, expert_weight, ln_gamma, ln_beta):
    b, s, h = hidden_states.shape
    e = gate_w.shape[0]
    m = b * s
    bm = min(256, m)
    m_blocks = m // bm

    x2d = hidden_states.reshape(m, h).astype(jnp.bfloat16)
    w16 = expert_weight.astype(jnp.bfloat16)
    gw16 = gate_w.astype(jnp.bfloat16)
    gamma2d = ln_gamma.reshape(1, h)
    beta2d = ln_beta.reshape(1, h)

    out = pl.pallas_call(
        functools.partial(_moe_kernel, num_experts=e),
        grid=(m_blocks,),
        in_specs=[
            pl.BlockSpec((bm, h), lambda i: (i, 0)),
            pl.BlockSpec((h, h), lambda i: (0, 0)),
            pl.BlockSpec((e, h), lambda i: (0, 0)),
            pl.BlockSpec((1, h), lambda i: (0, 0)),
            pl.BlockSpec((1, h), lambda i: (0, 0)),
        ],
        out_specs=pl.BlockSpec((bm, h), lambda i: (i, 0)),
        out_shape=jax.ShapeDtypeStruct((m, h), jnp.float32),
        compiler_params=pltpu.CompilerParams(
            dimension_semantics=("arbitrary",)),
    )(x2d, w16, gw16, gamma2d, beta2d)

    return out.reshape(b, s, h)


# speedup vs baseline: 1.2880x; 1.2117x over previous
import functools

import jax
import jax.numpy as jnp
from jax.experimental import pallas as pl
from jax.experimental.pallas import tpu as pltpu

_LN_EPS = 1e-5


def _moe_kernel(x_ref, w_ref, gw_ref, gamma_ref, beta_ref, o_ref,
                *, num_experts):
    x = x_ref[...].astype(jnp.bfloat16)
    acc = jnp.dot(x, w_ref[...], preferred_element_type=jnp.float32)
    logits = jax.lax.dot_general(
        x, gw_ref[...], (((1,), (1,)), ((), ())),
        preferred_element_type=jnp.float32)

    m1 = jnp.max(logits, axis=-1, keepdims=True)
    iota = jax.lax.broadcasted_iota(jnp.int32, logits.shape, 1)
    is_max = logits == m1
    first_idx = jnp.min(jnp.where(is_max, iota, num_experts),
                        axis=-1, keepdims=True)
    masked = jnp.where(iota == first_idx, -jnp.inf, logits)
    m2 = jnp.max(masked, axis=-1, keepdims=True)
    s = m1 + m2

    inv_h = 1.0 / acc.shape[-1]
    mu = jnp.sum(acc, axis=-1, keepdims=True) * inv_h
    msq = jnp.sum(acc * acc, axis=-1, keepdims=True) * inv_h
    var = msq - mu * mu
    coef = s * jax.lax.rsqrt(s * s * var + _LN_EPS)
    t = acc * coef - mu * coef
    o_ref[...] = t * gamma_ref[...] + beta_ref[...]


@jax.jit
def kernel(hidden_states, gate_w, expert_weight, ln_gamma, ln_beta):
    b, s, h = hidden_states.shape
    e = gate_w.shape[0]
    m = b * s
    bm = min(256, m)
    m_blocks = m // bm

    x2d = hidden_states.reshape(m, h)
    w16 = expert_weight.astype(jnp.bfloat16)
    gw16 = gate_w.astype(jnp.bfloat16)
    gamma2d = ln_gamma.reshape(1, h)
    beta2d = ln_beta.reshape(1, h)

    out = pl.pallas_call(
        functools.partial(_moe_kernel, num_experts=e),
        grid=(m_blocks,),
        in_specs=[
            pl.BlockSpec((bm, h), lambda i: (i, 0)),
            pl.BlockSpec((h, h), lambda i: (0, 0)),
            pl.BlockSpec((e, h), lambda i: (0, 0)),
            pl.BlockSpec((1, h), lambda i: (0, 0)),
            pl.BlockSpec((1, h), lambda i: (0, 0)),
        ],
        out_specs=pl.BlockSpec((bm, h), lambda i: (i, 0)),
        out_shape=jax.ShapeDtypeStruct((m, h), jnp.float32),
        compiler_params=pltpu.CompilerParams(
            dimension_semantics=("arbitrary",)),
    )(x2d, w16, gw16, gamma2d, beta2d)

    return out.reshape(b, s, h)
